# baseline (speedup 1.0000x reference)
import jax
import jax.numpy as jnp
from jax import lax
from jax.experimental import pallas as pl
from jax.experimental.pallas import tpu as pltpu
from jax.experimental.pallas import tpu_sc as plsc

_NC = 2
_NS = 16
_NW = _NC * _NS
_L = 16
_NSLOT = 4


def _make_embed(B, S, V, D):
    PW = S // _NW
    HC = PW // _NSLOT
    NPAIR = (B // 2) * _NSLOT

    mesh = plsc.VectorSubcoreMesh(
        core_axis_name="c", subcore_axis_name="s",
        num_cores=_NC, num_subcores=_NS,
    )

    def body(data_h, wte_h, wpe_h, out_h, idx_all, wpe_buf, *bufs_and_sems):
        gbufs = bufs_and_sems[0:2 * _NSLOT]
        obufs = bufs_and_sems[2 * _NSLOT:4 * _NSLOT]
        isem = bufs_and_sems[4 * _NSLOT]
        gsems = bufs_and_sems[4 * _NSLOT + 1:6 * _NSLOT + 1]
        osems = bufs_and_sems[6 * _NSLOT + 1:8 * _NSLOT + 1]

        wid = lax.axis_index("s") * _NC + lax.axis_index("c")
        p0 = wid * PW

        for b in range(B):
            pltpu.async_copy(
                data_h.at[pl.ds(b * S + p0, PW)], idx_all.at[b], isem)
        for b in range(B):
            pltpu.make_async_copy(
                data_h.at[pl.ds(b * S + p0, PW)], idx_all.at[b], isem).wait()

        pltpu.sync_copy(wpe_h.at[pl.ds(p0, PW)], wpe_buf)

        def pair_coords(t):
            q = lax.rem(t, _NSLOT)
            b0 = (t // _NSLOT) * 2
            roff = q * HC
            return b0, roff

        def start_gathers(t, s):
            b0, roff = pair_coords(t)
            pltpu.async_copy(
                wte_h.at[idx_all.at[b0, pl.ds(roff, HC)]],
                gbufs[2 * s], gsems[2 * s])
            pltpu.async_copy(
                wte_h.at[idx_all.at[b0 + 1, pl.ds(roff, HC)]],
                gbufs[2 * s + 1], gsems[2 * s + 1])

        for s in range(_NSLOT):
            start_gathers(s, s)

        def step(g, carry):
            for s in range(_NSLOT):
                gx, gy = gbufs[2 * s], gbufs[2 * s + 1]
                ox, oy = obufs[2 * s], obufs[2 * s + 1]
                gsx, gsy = gsems[2 * s], gsems[2 * s + 1]
                osx, osy = osems[2 * s], osems[2 * s + 1]
                t = _NSLOT * g + s
                b0, roff = pair_coords(t)

                pltpu.make_async_copy(
                    wte_h.at[idx_all.at[b0, pl.ds(roff, HC)]], gx, gsx).wait()
                pltpu.make_async_copy(
                    wte_h.at[idx_all.at[b0 + 1, pl.ds(roff, HC)]], gy, gsy).wait()

                @pl.when(g > 0)
                def _():
                    pltpu.make_async_copy(
                        ox, out_h.at[b0, pl.ds(p0 + roff, HC)], osx).wait()
                    pltpu.make_async_copy(
                        oy, out_h.at[b0 + 1, pl.ds(p0 + roff, HC)], osy).wait()

                def addrow(r, c_):
                    for c in range(D // _L):
                        sl = pl.ds(c * _L, _L)
                        w = wpe_buf[roff + r, sl]
                        ox[r, sl] = gx[r, sl] + w
                        oy[r, sl] = gy[r, sl] + w
                    return c_
                lax.fori_loop(0, HC, addrow, 0)

                @pl.when(t + _NSLOT < NPAIR)
                def _():
                    start_gathers(t + _NSLOT, s)

                pltpu.async_copy(
                    ox, out_h.at[b0, pl.ds(p0 + roff, HC)], osx)
                pltpu.async_copy(
                    oy, out_h.at[b0 + 1, pl.ds(p0 + roff, HC)], osy)
            return carry

        lax.fori_loop(0, NPAIR // _NSLOT, step, 0)

        for s in range(_NSLOT):
            t = NPAIR - _NSLOT + s
            b0, roff = pair_coords(t)
            pltpu.make_async_copy(
                obufs[2 * s], out_h.at[b0, pl.ds(p0 + roff, HC)],
                osems[2 * s]).wait()
            pltpu.make_async_copy(
                obufs[2 * s + 1], out_h.at[b0 + 1, pl.ds(p0 + roff, HC)],
                osems[2 * s + 1]).wait()

    return pl.kernel(
        body,
        out_type=jax.ShapeDtypeStruct((B, S, D), jnp.float32),
        mesh=mesh,
        scratch_types=(
            [pltpu.VMEM((B, PW), jnp.int32),
             pltpu.VMEM((PW, D), jnp.float32)]
            + [pltpu.VMEM((HC, D), jnp.float32) for _ in range(2 * _NSLOT)]
            + [pltpu.VMEM((HC, D), jnp.float32) for _ in range(2 * _NSLOT)]
            + [pltpu.SemaphoreType.DMA for _ in range(4 * _NSLOT + 1)]
        ),
    )


def kernel(data, wte, wpe):
    B, S = data.shape
    V, D = wte.shape
    embed = _make_embed(B, S, V, D)
    return embed(data.astype(jnp.int32).reshape(B * S), wte, wpe)


# speedup vs baseline: 1.0329x; 1.0329x over previous
import jax
import jax.numpy as jnp
from jax import lax
from jax.experimental import pallas as pl
from jax.experimental.pallas import tpu as pltpu
from jax.experimental.pallas import tpu_sc as plsc

_NC = 2
_NS = 16
_NW = _NC * _NS
_L = 16


def _make_embed(B, S, V, D):
    PW = S // _NW
    HC = PW // 2
    NPAIR = B

    mesh = plsc.VectorSubcoreMesh(
        core_axis_name="c", subcore_axis_name="s",
        num_cores=_NC, num_subcores=_NS,
    )

    def body(data_h, wte_h, wpe_h, out_h,
             idx_all, wpe_buf,
             gax, gay, gbx, gby, oax, oay, obx, oby,
             isem, wsem, gsax, gsay, gsbx, gsby, osax, osay, osbx, osby):
        wid = lax.axis_index("s") * _NC + lax.axis_index("c")
        p0 = wid * PW

        slots = ((gax, gay, oax, oay, gsax, gsay, osax, osay),
                 (gbx, gby, obx, oby, gsbx, gsby, osbx, osby))

        def pair_coords(t):
            h = lax.rem(t, 2)
            b0 = (t // 2) * 2
            roff = h * HC
            return b0, roff

        def start_gathers(t, gx, gy, gsx, gsy):
            b0, roff = pair_coords(t)
            pltpu.async_copy(
                wte_h.at[idx_all.at[b0, pl.ds(roff, HC)]], gx, gsx)
            pltpu.async_copy(
                wte_h.at[idx_all.at[b0 + 1, pl.ds(roff, HC)]], gy, gsy)

        for b in range(2):
            pltpu.async_copy(
                data_h.at[pl.ds(b * S + p0, PW)], idx_all.at[b], isem)
        pltpu.async_copy(wpe_h.at[pl.ds(p0, PW)], wpe_buf, wsem)
        for b in range(2, B):
            pltpu.async_copy(
                data_h.at[pl.ds(b * S + p0, PW)], idx_all.at[b], isem)
        for b in range(2):
            pltpu.make_async_copy(
                data_h.at[pl.ds(b * S + p0, PW)], idx_all.at[b], isem).wait()

        for s in range(2):
            gx, gy, _, _, gsx, gsy, _, _ = slots[s]
            start_gathers(s, gx, gy, gsx, gsy)

        for b in range(2, B):
            pltpu.make_async_copy(
                data_h.at[pl.ds(b * S + p0, PW)], idx_all.at[b], isem).wait()
        pltpu.make_async_copy(wpe_h.at[pl.ds(p0, PW)], wpe_buf, wsem).wait()

        def step(g, carry):
            for s in range(2):
                gx, gy, ox, oy, gsx, gsy, osx, osy = slots[s]
                t = 2 * g + s
                b0, roff = pair_coords(t)

                pltpu.make_async_copy(
                    wte_h.at[idx_all.at[b0, pl.ds(roff, HC)]], gx, gsx).wait()
                pltpu.make_async_copy(
                    wte_h.at[idx_all.at[b0 + 1, pl.ds(roff, HC)]], gy, gsy).wait()

                @pl.when(g > 0)
                def _():
                    pltpu.make_async_copy(
                        ox, out_h.at[b0, pl.ds(p0 + roff, HC)], osx).wait()
                    pltpu.make_async_copy(
                        oy, out_h.at[b0 + 1, pl.ds(p0 + roff, HC)], osy).wait()

                def addrow(r, c_):
                    for c in range(D // _L):
                        sl = pl.ds(c * _L, _L)
                        w = wpe_buf[roff + r, sl]
                        ox[r, sl] = gx[r, sl] + w
                        oy[r, sl] = gy[r, sl] + w
                    return c_
                lax.fori_loop(0, HC, addrow, 0)

                @pl.when(t + 2 < NPAIR)
                def _():
                    start_gathers(t + 2, gx, gy, gsx, gsy)

                pltpu.async_copy(
                    ox, out_h.at[b0, pl.ds(p0 + roff, HC)], osx)
                pltpu.async_copy(
                    oy, out_h.at[b0 + 1, pl.ds(p0 + roff, HC)], osy)
            return carry

        lax.fori_loop(0, NPAIR // 2, step, 0)

        for s in range(2):
            _, _, ox, oy, _, _, osx, osy = slots[s]
            t = NPAIR - 2 + s
            b0, roff = pair_coords(t)
            pltpu.make_async_copy(
                ox, out_h.at[b0, pl.ds(p0 + roff, HC)], osx).wait()
            pltpu.make_async_copy(
                oy, out_h.at[b0 + 1, pl.ds(p0 + roff, HC)], osy).wait()

    return pl.kernel(
        body,
        out_type=jax.ShapeDtypeStruct((B, S, D), jnp.float32),
        mesh=mesh,
        scratch_types=[
            pltpu.VMEM((B, PW), jnp.int32),
            pltpu.VMEM((PW, D), jnp.float32),
            pltpu.VMEM((HC, D), jnp.float32),
            pltpu.VMEM((HC, D), jnp.float32),
            pltpu.VMEM((HC, D), jnp.float32),
            pltpu.VMEM((HC, D), jnp.float32),
            pltpu.VMEM((HC, D), jnp.float32),
            pltpu.VMEM((HC, D), jnp.float32),
            pltpu.VMEM((HC, D), jnp.float32),
            pltpu.VMEM((HC, D), jnp.float32),
            pltpu.SemaphoreType.DMA,
            pltpu.SemaphoreType.DMA,
            pltpu.SemaphoreType.DMA,
            pltpu.SemaphoreType.DMA,
            pltpu.SemaphoreType.DMA,
            pltpu.SemaphoreType.DMA,
            pltpu.SemaphoreType.DMA,
            pltpu.SemaphoreType.DMA,
            pltpu.SemaphoreType.DMA,
            pltpu.SemaphoreType.DMA,
        ],
    )


def kernel(data, wte, wpe):
    B, S = data.shape
    V, D = wte.shape
    embed = _make_embed(B, S, V, D)
    return embed(data.astype(jnp.int32).reshape(B * S), wte, wpe)
